# baseline (speedup 1.0000x reference)
import functools

import jax
import jax.numpy as jnp
from jax import lax
from jax.experimental import pallas as pl
from jax.experimental.pallas import tpu as pltpu
from jax.experimental.pallas import tpu_sc as plsc

N_ATOMS = 10000
N_PAIRS = 320000
NB = 128
NR = 64
NBLK = 5
NRI = 3
NRF = 2

_LOG2 = 0.6931471805599453

_NC = 2
_NS = 16
_NW = _NC * _NS
_E_PER_W = N_PAIRS // _NW
_K = 80
_CHUNKS = _E_PER_W // _K
_QUADS = _CHUNKS // 4
_RPART = 624
_ZB = 8
_ZROWS = 208


def _ssp(v):
    return jnp.maximum(v, 0.0) + jnp.log1p(jnp.exp(-jnp.abs(v))) - _LOG2


def _node_in_body(x_ref, wi_ref, bi_ref, wj_ref, bj_ref, xi_ref, xjd_ref):
    xa = _ssp(x_ref[...])
    xi_ref[...] = _ssp(
        jnp.dot(xa, wi_ref[...], preferred_element_type=jnp.float32) + bi_ref[...])
    xjd_ref[...] = _ssp(
        jnp.dot(xa, wj_ref[...], preferred_element_type=jnp.float32) + bj_ref[...])


_NODE_TILE = 2000


def _node_in(x, wi, bi, wj, bj):
    grid = (N_ATOMS // _NODE_TILE,)
    row = lambda i: (i, 0)
    full = lambda i: (0, 0)
    return pl.pallas_call(
        _node_in_body,
        grid=grid,
        in_specs=[
            pl.BlockSpec((_NODE_TILE, NB), row),
            pl.BlockSpec((NB, NB), full),
            pl.BlockSpec((1, NB), full),
            pl.BlockSpec((NB, NB), full),
            pl.BlockSpec((1, NB), full),
        ],
        out_specs=[
            pl.BlockSpec((_NODE_TILE, NB), row),
            pl.BlockSpec((_NODE_TILE, NB), row),
        ],
        out_shape=[
            jax.ShapeDtypeStruct((N_ATOMS, NB), jnp.float32),
            jax.ShapeDtypeStruct((N_ATOMS, NB), jnp.float32),
        ],
    )(x, wi, bi, wj, bj)


_EDGE_TILE = 2000


def _g_body(c_ref, rbf_ref, wg_ref, g_ref):
    desc = c_ref[...] * rbf_ref[...]
    g_ref[...] = jnp.dot(desc, wg_ref[...], preferred_element_type=jnp.float32)


def _g_matmul(cutoffs2d, rbfs, wg):
    grid = (N_PAIRS // _EDGE_TILE,)
    row = lambda i: (i, 0)
    full = lambda i: (0, 0)
    return pl.pallas_call(
        _g_body,
        grid=grid,
        in_specs=[
            pl.BlockSpec((_EDGE_TILE, 1), row),
            pl.BlockSpec((_EDGE_TILE, NR), row),
            pl.BlockSpec((NR, NB), full),
        ],
        out_specs=pl.BlockSpec((_EDGE_TILE, NB), row),
        out_shape=jax.ShapeDtypeStruct((N_PAIRS, NB), jnp.float32),
    )(cutoffs2d, rbfs, wg)


def _sc_agg_body(g_hbm, xjd_hbm, idxi_hbm, idxj_hbm, out_hbm,
                 aggsh, idxi_v, idxj_v, gp_v, rows_v,
                 sem_i0, sem_i1, sem_i2, sem_i3, sem_g0, sem_g1,
                 sem_r0, sem_r1, sem_s0, sem_s1):
    cid = lax.axis_index("c")
    sid = lax.axis_index("s")
    wid = cid * _NS + sid
    sem_i = (sem_i0, sem_i1, sem_i2, sem_i3)
    sem_g = (sem_g0, sem_g1)
    sem_r = (sem_r0, sem_r1)
    sem_s = (sem_s0, sem_s1)

    def zloop(i, carry):
        for c in range(NB // 16):
            gp_v[0, i, pl.ds(c * 16, 16)] = jnp.zeros((16,), jnp.float32)
        return carry
    lax.fori_loop(0, 8, zloop, 0)

    def zcopy(t, carry):
        pltpu.sync_copy(gp_v.at[0, pl.ds(0, 8)],
                        aggsh.at[pl.ds(sid * _RPART + t * 8, 8)])
        return carry
    lax.fori_loop(0, _RPART // 8, zcopy, 0)

    @pl.when(sid == _NS - 1)
    def _zero_tail():
        pltpu.sync_copy(gp_v.at[0, pl.ds(0, 8)],
                        aggsh.at[pl.ds(_RPART * _NS, 8)])
        pltpu.sync_copy(gp_v.at[0, pl.ds(0, 8)],
                        aggsh.at[pl.ds(_RPART * _NS + 8, 8)])
    plsc.subcore_barrier()

    base0 = wid * _E_PER_W

    def _idx_loads(ck, s4):
        base = base0 + ck * _K
        pltpu.async_copy(idxj_hbm.at[pl.ds(base, _K)], idxj_v.at[s4], sem_i[s4])
        pltpu.async_copy(idxi_hbm.at[pl.ds(base, _K)], idxi_v.at[s4], sem_i[s4])

    def _wait_idx(ck, s4):
        base = base0 + ck * _K
        pltpu.make_async_copy(idxj_hbm.at[pl.ds(base, _K)], idxj_v.at[s4],
                              sem_i[s4]).wait()
        pltpu.make_async_copy(idxi_hbm.at[pl.ds(base, _K)], idxi_v.at[s4],
                              sem_i[s4]).wait()

    def _gp_load(ck, s2):
        base = base0 + ck * _K
        pltpu.async_copy(g_hbm.at[pl.ds(base, _K)], gp_v.at[s2], sem_g[s2])

    def _wait_gp(ck, s2):
        base = base0 + ck * _K
        pltpu.make_async_copy(g_hbm.at[pl.ds(base, _K)], gp_v.at[s2],
                              sem_g[s2]).wait()

    def _gather(s4, s2):
        pltpu.async_copy(xjd_hbm.at[idxj_v.at[s4]], rows_v.at[s2], sem_r[s2])

    def _wait_gather(s4, s2):
        pltpu.make_async_copy(xjd_hbm.at[idxj_v.at[s4]], rows_v.at[s2],
                              sem_r[s2]).wait()

    def _scatter(s4, s2):
        pltpu.async_copy(gp_v.at[s2], aggsh.at[idxi_v.at[s4]], sem_s[s2],
                         add=True)

    def _wait_scatter(s4, s2):
        pltpu.make_async_copy(gp_v.at[s2], aggsh.at[idxi_v.at[s4]],
                              sem_s[s2]).wait()

    def _compute(s2):
        def egroup(eg, carry):
            e0 = eg * 8
            for de in range(8):
                e = e0 + de
                for c in range(NB // 16):
                    s = pl.ds(c * 16, 16)
                    gp_v[s2, e, s] = gp_v[s2, e, s] * rows_v[s2, e, s]
            return carry
        lax.fori_loop(0, _K // 8, egroup, 0)

    _idx_loads(0, 0)
    _idx_loads(1, 1)
    _gp_load(0, 0)
    _wait_idx(0, 0)
    _gather(0, 0)

    def quad(q, carry):
        for b in range(4):
            ck = 4 * q + b
            s2 = b % 2
            _wait_idx(ck + 1, (b + 1) % 4)
            _gather((b + 1) % 4, 1 - s2)
            if b == 0:
                @pl.when(q > 0)
                def _ws():
                    _wait_scatter(3, 1)
            else:
                _wait_scatter(b - 1, 1 - s2)
            _gp_load(ck + 1, 1 - s2)
            _wait_gather(b, s2)
            _wait_gp(ck, s2)
            _compute(s2)
            _scatter(b, s2)
            if b == 3:
                @pl.when(q < _QUADS - 1)
                def _il():
                    _idx_loads(ck + 2, (b + 2) % 4)
            else:
                _idx_loads(ck + 2, (b + 2) % 4)
        return carry

    lax.fori_loop(0, _QUADS, quad, 0)

    _wait_scatter(3, 1)
    _wait_gather(0, 0)
    _wait_gp(_CHUNKS - 1, 0)
    _compute(0)
    _scatter(0, 0)
    _wait_scatter(0, 0)
    plsc.subcore_barrier()

    for t in range(_RPART // _ZROWS):
        r0 = sid * _RPART + t * _ZROWS
        pltpu.sync_copy(aggsh.at[pl.ds(r0, _ZROWS)],
                        out_hbm.at[cid, pl.ds(r0, _ZROWS)])

    @pl.when(sid == _NS - 1)
    def _copy_tail():
        pltpu.sync_copy(aggsh.at[pl.ds(_RPART * _NS, 16)],
                        out_hbm.at[cid, pl.ds(_RPART * _NS, 16)])


def _sc_agg(g, xjd, idx_i, idx_j):
    mesh = plsc.VectorSubcoreMesh(core_axis_name="c", subcore_axis_name="s")
    kfn = functools.partial(
        pl.kernel,
        mesh=mesh,
        out_type=jax.ShapeDtypeStruct((_NC, N_ATOMS, NB), jnp.float32),
        scratch_types=[
            pltpu.VMEM_SHARED((N_ATOMS, NB), jnp.float32),
            pltpu.VMEM((4, _K), jnp.int32),
            pltpu.VMEM((4, _K), jnp.int32),
            pltpu.VMEM((2, _K, NB), jnp.float32),
            pltpu.VMEM((2, _K, NB), jnp.float32),
        ] + [pltpu.SemaphoreType.DMA] * 10,
    )(_sc_agg_body)
    return kfn(g, xjd, idx_i, idx_j)


def _node_out_body(x_ref, xi_ref, a0_ref, a1_ref, wri_ref, bri_ref,
                   wout_ref, bout_ref, u_ref, wrf_ref, brf_ref, o_ref):
    m = xi_ref[...] + a0_ref[...] + a1_ref[...]
    for r in range(NRI):
        y = _ssp(m)
        y = _ssp(jnp.dot(y, wri_ref[2 * r], preferred_element_type=jnp.float32)
                 + bri_ref[2 * r])
        y = (jnp.dot(y, wri_ref[2 * r + 1], preferred_element_type=jnp.float32)
             + bri_ref[2 * r + 1])
        m = m + y
    m = _ssp(m)
    x = (u_ref[...] * x_ref[...]
         + jnp.dot(m, wout_ref[...], preferred_element_type=jnp.float32)
         + bout_ref[...])
    for r in range(NRF):
        y = _ssp(x)
        y = _ssp(jnp.dot(y, wrf_ref[2 * r], preferred_element_type=jnp.float32)
                 + brf_ref[2 * r])
        y = (jnp.dot(y, wrf_ref[2 * r + 1], preferred_element_type=jnp.float32)
             + brf_ref[2 * r + 1])
        x = x + y
    o_ref[...] = x


def _node_out(x, xi, a0, a1, wri, bri, wout, bout, u, wrf, brf):
    grid = (N_ATOMS // _NODE_TILE,)
    row = lambda i: (i, 0)
    full2 = lambda i: (0, 0)
    full3 = lambda i: (0, 0, 0)
    return pl.pallas_call(
        _node_out_body,
        grid=grid,
        in_specs=[
            pl.BlockSpec((_NODE_TILE, NB), row),
            pl.BlockSpec((_NODE_TILE, NB), row),
            pl.BlockSpec((_NODE_TILE, NB), row),
            pl.BlockSpec((_NODE_TILE, NB), row),
            pl.BlockSpec((2 * NRI, NB, NB), full3),
            pl.BlockSpec((2 * NRI, NB), full2),
            pl.BlockSpec((NB, NB), full2),
            pl.BlockSpec((1, NB), full2),
            pl.BlockSpec((1, NB), full2),
            pl.BlockSpec((2 * NRF, NB, NB), full3),
            pl.BlockSpec((2 * NRF, NB), full2),
        ],
        out_specs=pl.BlockSpec((_NODE_TILE, NB), row),
        out_shape=jax.ShapeDtypeStruct((N_ATOMS, NB), jnp.float32),
    )(x, xi, a0, a1, wri, bri, wout, bout, u, wrf, brf)


def kernel(features, distances, cutoffs, rbfs, idx_i, idx_j,
           Wg, Wi, bi, Wj, bj, Wri, bri, Wout, bout, u, Wrf, brf):
    del distances
    cutoffs2d = cutoffs.reshape(N_PAIRS, 1)
    idx_i32 = idx_i.astype(jnp.int32)
    idx_j32 = idx_j.astype(jnp.int32)
    bi2 = bi.reshape(NBLK, 1, NB)
    bj2 = bj.reshape(NBLK, 1, NB)
    bri2 = bri.reshape(NBLK, 2 * NRI, NB)
    wri2 = Wri.reshape(NBLK, 2 * NRI, NB, NB)
    bout2 = bout.reshape(NBLK, 1, NB)
    u2 = u.reshape(NBLK, 1, NB)
    brf2 = brf.reshape(NBLK, 2 * NRF, NB)
    wrf2 = Wrf.reshape(NBLK, 2 * NRF, NB, NB)

    x = features
    outs = []
    gs = [_g_matmul(cutoffs2d, rbfs, Wg[k]) for k in range(NBLK)]
    for k in range(NBLK):
        xi, xjd = _node_in(x, Wi[k], bi2[k], Wj[k], bj2[k])
        agg = _sc_agg(gs[k], xjd, idx_i32, idx_j32)
        x = _node_out(x, xi, agg[0], agg[1], wri2[k], bri2[k],
                      Wout[k], bout2[k], u2[k], wrf2[k], brf2[k])
        outs.append(x)
    return jnp.stack(outs)
